# baseline (speedup 1.0000x reference)
import functools

import jax
import jax.numpy as jnp
from jax import lax
from jax.experimental import pallas as pl
from jax.experimental.pallas import tpu as pltpu
from jax.experimental.pallas import tpu_sc as plsc

NC = 2
NS = 16
NW = NC * NS
EMBED = 64
LANES = 16

BATCH = 16384
HIST = 50
COLS_PER_W = BATCH // NW
JC = 256
JOBS_PER_H = COLS_PER_W // JC
NJOBS = HIST * JOBS_PER_H


def _gather_body(idx_hbm, tab_hbm, out_hbm, idx_v, ga, gb, ta, tb, gsa, gsb, wsa, wsb):
    sid = lax.axis_index("s")
    cid = lax.axis_index("c")
    wid = sid * NC + cid
    c0 = wid * COLS_PER_W
    pltpu.sync_copy(idx_hbm.at[:, pl.ds(c0, COLS_PER_W)], idx_v)

    lane = lax.iota(jnp.int32, LANES)

    def fire_gather(j, gbuf, sem):
        h = j // JOBS_PER_H
        half = j % JOBS_PER_H
        pltpu.async_copy(tab_hbm.at[idx_v.at[h, pl.ds(half * JC, JC)]], gbuf, sem)

    def wait_gather(gbuf, sem):
        pltpu.make_async_copy(tab_hbm.at[pl.ds(0, JC)], gbuf, sem).wait()

    def transpose(gbuf, tbuf):
        @pl.loop(0, EMBED)
        def _(d):
            col = jnp.full((LANES,), d, jnp.int32)
            for k in range(JC // LANES):
                rows = lane + (k * LANES)
                v = plsc.load_gather(gbuf, [rows, col])
                tbuf[d, pl.ds(k * LANES, LANES)] = v

    def fire_write(j, tbuf, sem):
        h = j // JOBS_PER_H
        half = j % JOBS_PER_H
        pltpu.async_copy(
            tbuf, out_hbm.at[h, :, pl.ds(c0 + half * JC, JC)], sem
        )

    def wait_write(tbuf, sem):
        pltpu.make_async_copy(tbuf, out_hbm.at[0, :, pl.ds(c0, JC)], sem).wait()

    fire_gather(0, ga, gsa)

    @pl.loop(0, NJOBS, step=2)
    def _(j):
        fire_gather(j + 1, gb, gsb)
        wait_gather(ga, gsa)
        pl.when(j > 0)(lambda: wait_write(ta, wsa))
        transpose(ga, ta)
        fire_write(j, ta, wsa)
        pl.when(j + 2 < NJOBS)(lambda: fire_gather(j + 2, ga, gsa))
        wait_gather(gb, gsb)
        pl.when(j > 0)(lambda: wait_write(tb, wsb))
        transpose(gb, tb)
        fire_write(j + 1, tb, wsb)

    wait_write(ta, wsa)
    wait_write(tb, wsb)


_gather = functools.partial(
    pl.kernel,
    out_type=jax.ShapeDtypeStruct((HIST, EMBED, BATCH), jnp.float32),
    mesh=plsc.VectorSubcoreMesh(
        core_axis_name="c", subcore_axis_name="s", num_cores=NC, num_subcores=NS
    ),
    scratch_types=[
        pltpu.VMEM((HIST, COLS_PER_W), jnp.int32),
        pltpu.VMEM((JC, EMBED), jnp.float32),
        pltpu.VMEM((JC, EMBED), jnp.float32),
        pltpu.VMEM((EMBED, JC), jnp.float32),
        pltpu.VMEM((EMBED, JC), jnp.float32),
        pltpu.SemaphoreType.DMA,
        pltpu.SemaphoreType.DMA,
        pltpu.SemaphoreType.DMA,
        pltpu.SemaphoreType.DMA,
    ],
    compiler_params=pltpu.CompilerParams(
        use_tc_tiling_on_sc=False, needs_layout_passes=False
    ),
)(_gather_body)


def kernel(input, weight):
    idx_t = jnp.transpose(input.astype(jnp.int32))
    out_t = _gather(idx_t, weight)
    return jnp.transpose(out_t, (2, 0, 1))
